# baseline (speedup 1.0000x reference)
import functools
import math

import jax
import jax.numpy as jnp
from jax import lax
from jax.experimental import pallas as pl
from jax.experimental.pallas import tpu as pltpu
from jax.experimental.pallas import tpu_sc as plsc

UNITS = 100000
NUM_SAMPLED = 8192
BATCH = 1024
DIM = 16
TOTAL_IDS = NUM_SAMPLED + BATCH
NUM_WORKERS = 32
PER_W = TOTAL_IDS // NUM_WORKERS
CHUNKS = 3
CHUNK = PER_W // CHUNKS
LANE = 16
BC = 2048
TBC = 16384
LOG_RANGE = math.log(float(UNITS) + 1.0)


def _sampled_ids_and_adj():
    key = jax.random.key(42)
    u = jax.random.uniform(key, (NUM_SAMPLED,), dtype=jnp.float32)
    s = jnp.exp(u * jnp.log(float(UNITS) + 1.0)) - 1.0
    ids = jnp.clip(s.astype(jnp.int32), 0, UNITS - 1)
    idf = ids.astype(jnp.float32)
    p = (jnp.log(idf + 2.0) - jnp.log(idf + 1.0)) / LOG_RANGE
    adj = jnp.log(-jnp.expm1(float(NUM_SAMPLED) * jnp.log1p(-p)))
    return ids, adj


def _transpad_body(k_ref, eye_ref, out_ref):
    out_ref[...] = lax.dot_general(
        k_ref[...], eye_ref[...], (((0,), (0,)), ((), ())),
        preferred_element_type=jnp.float32)


def _transpad(kern, eye):
    return pl.pallas_call(
        _transpad_body,
        grid=(pl.cdiv(UNITS, TBC),),
        in_specs=[
            pl.BlockSpec((DIM, TBC), lambda j: (0, j)),
            pl.BlockSpec((DIM, 128), lambda j: (0, 0)),
        ],
        out_specs=pl.BlockSpec((TBC, 128), lambda j: (j, 0)),
        out_shape=jax.ShapeDtypeStruct((UNITS, 128), jnp.float32),
    )(kern, eye)


def _sc_gather(table, bias, idx, idx8, adj):
    mesh = plsc.VectorSubcoreMesh(core_axis_name="c", subcore_axis_name="s")

    @functools.partial(
        pl.kernel,
        mesh=mesh,
        out_type=(jax.ShapeDtypeStruct((TOTAL_IDS, DIM), jnp.float32),
                  jax.ShapeDtypeStruct((TOTAL_IDS,), jnp.float32)),
        name="sc_gather",
        scratch_types=[
            pltpu.VMEM((CHUNKS, CHUNK), jnp.int32),
            pltpu.VMEM((CHUNKS, CHUNK), jnp.int32),
            pltpu.VMEM((CHUNKS, CHUNK, DIM), jnp.float32),
            pltpu.VMEM((CHUNKS, CHUNK), jnp.float32),
            pltpu.VMEM((CHUNKS, CHUNK), jnp.float32),
            pltpu.SemaphoreType.DMA,
            pltpu.SemaphoreType.DMA,
        ],
        compiler_params=pltpu.CompilerParams(use_tc_tiling_on_sc=False),
    )
    def gather_kernel(table_hbm, bias_hbm, idx_hbm, idx8_hbm, adj_hbm,
                      out_hbm, bout_hbm,
                      idx_v, idx8_v, rows_v, bias_v, adj_v, isem, sem):
        wid = lax.axis_index("s") * 2 + lax.axis_index("c")
        base = wid * PER_W
        icopies = [pltpu.async_copy(idx_hbm.at[pl.ds(base + j * CHUNK, CHUNK)],
                                    idx_v.at[j], isem) for j in range(CHUNKS)]
        icopies += [pltpu.async_copy(idx8_hbm.at[pl.ds(base + j * CHUNK, CHUNK)],
                                     idx8_v.at[j], isem) for j in range(CHUNKS)]
        icopies += [pltpu.async_copy(adj_hbm.at[pl.ds(base + j * CHUNK, CHUNK)],
                                     adj_v.at[j], isem) for j in range(CHUNKS)]
        for c in icopies:
            c.wait()
        gathers = [pltpu.async_copy(table_hbm.at[idx8_v.at[j]], rows_v.at[j], sem)
                   for j in range(CHUNKS)]
        gathers += [pltpu.async_copy(bias_hbm.at[idx_v.at[j]], bias_v.at[j], sem)
                    for j in range(CHUNKS)]
        for c in gathers:
            c.wait()
        for j in range(CHUNKS):
            for k in range(CHUNK // LANE):
                sl = pl.ds(k * LANE, LANE)
                bias_v[j, sl] = bias_v[j, sl] - adj_v[j, sl]
        wcopies = [pltpu.async_copy(rows_v.at[j],
                                    out_hbm.at[pl.ds(base + j * CHUNK, CHUNK)],
                                    isem) for j in range(CHUNKS)]
        wcopies += [pltpu.async_copy(bias_v.at[j],
                                     bout_hbm.at[pl.ds(base + j * CHUNK, CHUNK)],
                                     isem) for j in range(CHUNKS)]
        for c in wcopies:
            c.wait()

    return gather_kernel(table, bias, idx, idx8, adj)


def _mm_body(k_ref, b_ref, pt_ref, out_ref):
    kb = jnp.concatenate([k_ref[...], b_ref[...]], axis=0)
    out_ref[...] = lax.dot_general(
        kb, pt_ref[...], (((0,), (0,)), ((), ())),
        preferred_element_type=jnp.float32)


def _projection_t(kern, bias_row, pred_t1):
    grid = (pl.cdiv(UNITS, BC),)
    return pl.pallas_call(
        _mm_body,
        grid=grid,
        in_specs=[
            pl.BlockSpec((DIM, BC), lambda j: (0, j)),
            pl.BlockSpec((1, BC), lambda j: (0, j)),
            pl.BlockSpec((DIM + 1, BATCH), lambda j: (0, 0)),
        ],
        out_specs=pl.BlockSpec((BC, BATCH), lambda j: (j, 0)),
        out_shape=jax.ShapeDtypeStruct((UNITS, BATCH), jnp.float32),
    )(kern, bias_row, pred_t1)


_N_SAMP_BLKS = NUM_SAMPLED // BATCH


def _loss_body(pred_ref, rows_ref, ba_ref, bt_ref, tgt_ref, out_ref, acc_ref):
    j = pl.program_id(0)

    @pl.when(j == 0)
    def _init():
        acc_ref[...] = jnp.zeros_like(acc_ref)

    @pl.when(j < _N_SAMP_BLKS)
    def _sampled():
        logits = lax.dot_general(pred_ref[...], rows_ref[...],
                                 (((1,), (1,)), ((), ())),
                                 preferred_element_type=jnp.float32)
        l = logits + ba_ref[0]
        ce = jnp.maximum(l, 0.0) + jnp.log1p(jnp.exp(-jnp.abs(l)))
        ones = jnp.ones((BATCH, 1), jnp.float32)
        acc_ref[...] += lax.dot_general(
            ce, ones, (((1,), (0,)), ((), ())),
            preferred_element_type=jnp.float32)

    @pl.when(j == _N_SAMP_BLKS)
    def _true():
        tl = jnp.sum(pred_ref[...] * rows_ref[...], axis=1,
                     keepdims=True) + bt_ref[...]
        t = tgt_ref[...]
        p = (jnp.log(t + 2.0) - jnp.log(t + 1.0)) / LOG_RANGE
        ec = 1.0 - jnp.exp(float(NUM_SAMPLED) * jnp.log1p(-p))
        l = tl - jnp.log(ec)
        ce1 = jnp.maximum(l, 0.0) - l + jnp.log1p(jnp.exp(-jnp.abs(l)))
        total = acc_ref[...] + ce1
        out_ref[...] = (jnp.sum(total) / float(BATCH)).reshape(1, 1)


def _nce_loss(pred, rows, ba9, btrue, tgtf):
    return pl.pallas_call(
        _loss_body,
        grid=(_N_SAMP_BLKS + 1,),
        in_specs=[
            pl.BlockSpec((BATCH, DIM), lambda j: (0, 0)),
            pl.BlockSpec((BATCH, DIM), lambda j: (j, 0)),
            pl.BlockSpec((1, 1, BATCH),
                         lambda j: (jnp.minimum(j, _N_SAMP_BLKS - 1), 0, 0)),
            pl.BlockSpec((BATCH, 1), lambda j: (0, 0)),
            pl.BlockSpec((BATCH, 1), lambda j: (0, 0)),
        ],
        out_specs=pl.BlockSpec((1, 1), lambda j: (0, 0)),
        out_shape=jax.ShapeDtypeStruct((1, 1), jnp.float32),
        scratch_shapes=[pltpu.VMEM((BATCH, 1), jnp.float32)],
    )(pred, rows, ba9, btrue, tgtf)


def kernel(pred, target, kernel, bias):
    sampled_ids, adj_s = _sampled_ids_and_adj()
    tgt = target.reshape(-1).astype(jnp.int32)

    ids = jnp.concatenate([sampled_ids, tgt])
    adj_ext = jnp.concatenate([adj_s, jnp.zeros((BATCH,), jnp.float32)])
    eye = jnp.eye(DIM, 128, dtype=jnp.float32)
    table_lin = _transpad(kernel, eye).reshape(8 * UNITS, DIM)
    rows, ba = _sc_gather(table_lin, bias, ids, ids * 8, adj_ext)

    pred_t1 = jnp.concatenate(
        [pred.T, jnp.ones((1, BATCH), jnp.float32)], axis=0)
    out_t = _projection_t(kernel, bias.reshape(1, UNITS), pred_t1)

    ba9 = ba[:NUM_SAMPLED].reshape(_N_SAMP_BLKS, 1, BATCH)
    btrue = ba[NUM_SAMPLED:].reshape(BATCH, 1)
    tgtf = tgt.astype(jnp.float32).reshape(BATCH, 1)
    loss = _nce_loss(pred, rows, ba9, btrue, tgtf)

    return (out_t.T, loss.reshape(()))


# speedup vs baseline: 1.0045x; 1.0045x over previous
import functools
import math

import jax
import jax.numpy as jnp
from jax import lax
from jax.experimental import pallas as pl
from jax.experimental.pallas import tpu as pltpu
from jax.experimental.pallas import tpu_sc as plsc

UNITS = 100000
NUM_SAMPLED = 8192
BATCH = 1024
DIM = 16
TOTAL_IDS = NUM_SAMPLED + BATCH
NUM_WORKERS = 32
PER_W = TOTAL_IDS // NUM_WORKERS
CHUNKS = 3
CHUNK = PER_W // CHUNKS
LANE = 16
BC = 6144
TBC = 16384
LOG_RANGE = math.log(float(UNITS) + 1.0)


def _sampled_ids_and_adj():
    key = jax.random.key(42)
    u = jax.random.uniform(key, (NUM_SAMPLED,), dtype=jnp.float32)
    s = jnp.exp(u * jnp.log(float(UNITS) + 1.0)) - 1.0
    ids = jnp.clip(s.astype(jnp.int32), 0, UNITS - 1)
    idf = ids.astype(jnp.float32)
    p = (jnp.log(idf + 2.0) - jnp.log(idf + 1.0)) / LOG_RANGE
    adj = jnp.log(-jnp.expm1(float(NUM_SAMPLED) * jnp.log1p(-p)))
    return ids, adj


def _transpad_body(k_ref, eye_ref, out_ref):
    out_ref[...] = lax.dot_general(
        k_ref[...], eye_ref[...], (((0,), (0,)), ((), ())),
        preferred_element_type=jnp.float32)


def _transpad(kern, eye):
    return pl.pallas_call(
        _transpad_body,
        grid=(pl.cdiv(UNITS, TBC),),
        in_specs=[
            pl.BlockSpec((DIM, TBC), lambda j: (0, j)),
            pl.BlockSpec((DIM, 128), lambda j: (0, 0)),
        ],
        out_specs=pl.BlockSpec((TBC, 128), lambda j: (j, 0)),
        out_shape=jax.ShapeDtypeStruct((UNITS, 128), jnp.float32),
    )(kern, eye)


def _sc_gather(table, bias, idx, idx8, adj):
    mesh = plsc.VectorSubcoreMesh(core_axis_name="c", subcore_axis_name="s")

    @functools.partial(
        pl.kernel,
        mesh=mesh,
        out_type=(jax.ShapeDtypeStruct((TOTAL_IDS, DIM), jnp.float32),
                  jax.ShapeDtypeStruct((TOTAL_IDS,), jnp.float32)),
        name="sc_gather",
        scratch_types=[
            pltpu.VMEM((CHUNKS, CHUNK), jnp.int32),
            pltpu.VMEM((CHUNKS, CHUNK), jnp.int32),
            pltpu.VMEM((CHUNKS, CHUNK, DIM), jnp.float32),
            pltpu.VMEM((CHUNKS, CHUNK), jnp.float32),
            pltpu.VMEM((CHUNKS, CHUNK), jnp.float32),
            pltpu.SemaphoreType.DMA,
            pltpu.SemaphoreType.DMA,
        ],
        compiler_params=pltpu.CompilerParams(use_tc_tiling_on_sc=False),
    )
    def gather_kernel(table_hbm, bias_hbm, idx_hbm, idx8_hbm, adj_hbm,
                      out_hbm, bout_hbm,
                      idx_v, idx8_v, rows_v, bias_v, adj_v, isem, sem):
        wid = lax.axis_index("s") * 2 + lax.axis_index("c")
        base = wid * PER_W
        icopies = [pltpu.async_copy(idx_hbm.at[pl.ds(base + j * CHUNK, CHUNK)],
                                    idx_v.at[j], isem) for j in range(CHUNKS)]
        icopies += [pltpu.async_copy(idx8_hbm.at[pl.ds(base + j * CHUNK, CHUNK)],
                                     idx8_v.at[j], isem) for j in range(CHUNKS)]
        icopies += [pltpu.async_copy(adj_hbm.at[pl.ds(base + j * CHUNK, CHUNK)],
                                     adj_v.at[j], isem) for j in range(CHUNKS)]
        for c in icopies:
            c.wait()
        gathers = [pltpu.async_copy(table_hbm.at[idx8_v.at[j]], rows_v.at[j], sem)
                   for j in range(CHUNKS)]
        gathers += [pltpu.async_copy(bias_hbm.at[idx_v.at[j]], bias_v.at[j], sem)
                    for j in range(CHUNKS)]
        for c in gathers:
            c.wait()
        for j in range(CHUNKS):
            for k in range(CHUNK // LANE):
                sl = pl.ds(k * LANE, LANE)
                bias_v[j, sl] = bias_v[j, sl] - adj_v[j, sl]
        wcopies = [pltpu.async_copy(rows_v.at[j],
                                    out_hbm.at[pl.ds(base + j * CHUNK, CHUNK)],
                                    isem) for j in range(CHUNKS)]
        wcopies += [pltpu.async_copy(bias_v.at[j],
                                     bout_hbm.at[pl.ds(base + j * CHUNK, CHUNK)],
                                     isem) for j in range(CHUNKS)]
        for c in wcopies:
            c.wait()

    return gather_kernel(table, bias, idx, idx8, adj)


def _mm_body(k_ref, b_ref, pt_ref, out_ref):
    kb = jnp.concatenate([k_ref[...], b_ref[...]], axis=0)
    out_ref[...] = lax.dot_general(
        kb, pt_ref[...], (((0,), (0,)), ((), ())),
        preferred_element_type=jnp.float32)


def _projection_t(kern, bias_row, pred_t1):
    grid = (pl.cdiv(UNITS, BC),)
    return pl.pallas_call(
        _mm_body,
        grid=grid,
        in_specs=[
            pl.BlockSpec((DIM, BC), lambda j: (0, j)),
            pl.BlockSpec((1, BC), lambda j: (0, j)),
            pl.BlockSpec((DIM + 1, BATCH), lambda j: (0, 0)),
        ],
        out_specs=pl.BlockSpec((BC, BATCH), lambda j: (j, 0)),
        out_shape=jax.ShapeDtypeStruct((UNITS, BATCH), jnp.float32),
    )(kern, bias_row, pred_t1)


_N_SAMP_BLKS = NUM_SAMPLED // BATCH


def _loss_body(pred_ref, rows_ref, ba_ref, bt_ref, tgt_ref, out_ref, acc_ref):
    j = pl.program_id(0)

    @pl.when(j == 0)
    def _init():
        acc_ref[...] = jnp.zeros_like(acc_ref)

    @pl.when(j < _N_SAMP_BLKS)
    def _sampled():
        logits = lax.dot_general(pred_ref[...], rows_ref[...],
                                 (((1,), (1,)), ((), ())),
                                 preferred_element_type=jnp.float32)
        l = logits + ba_ref[0]
        ce = jnp.maximum(l, 0.0) + jnp.log1p(jnp.exp(-jnp.abs(l)))
        ones = jnp.ones((BATCH, 1), jnp.float32)
        acc_ref[...] += lax.dot_general(
            ce, ones, (((1,), (0,)), ((), ())),
            preferred_element_type=jnp.float32)

    @pl.when(j == _N_SAMP_BLKS)
    def _true():
        tl = jnp.sum(pred_ref[...] * rows_ref[...], axis=1,
                     keepdims=True) + bt_ref[...]
        t = tgt_ref[...]
        p = (jnp.log(t + 2.0) - jnp.log(t + 1.0)) / LOG_RANGE
        ec = 1.0 - jnp.exp(float(NUM_SAMPLED) * jnp.log1p(-p))
        l = tl - jnp.log(ec)
        ce1 = jnp.maximum(l, 0.0) - l + jnp.log1p(jnp.exp(-jnp.abs(l)))
        total = acc_ref[...] + ce1
        out_ref[...] = (jnp.sum(total) / float(BATCH)).reshape(1, 1)


def _nce_loss(pred, rows, ba9, btrue, tgtf):
    return pl.pallas_call(
        _loss_body,
        grid=(_N_SAMP_BLKS + 1,),
        in_specs=[
            pl.BlockSpec((BATCH, DIM), lambda j: (0, 0)),
            pl.BlockSpec((BATCH, DIM), lambda j: (j, 0)),
            pl.BlockSpec((1, 1, BATCH),
                         lambda j: (jnp.minimum(j, _N_SAMP_BLKS - 1), 0, 0)),
            pl.BlockSpec((BATCH, 1), lambda j: (0, 0)),
            pl.BlockSpec((BATCH, 1), lambda j: (0, 0)),
        ],
        out_specs=pl.BlockSpec((1, 1), lambda j: (0, 0)),
        out_shape=jax.ShapeDtypeStruct((1, 1), jnp.float32),
        scratch_shapes=[pltpu.VMEM((BATCH, 1), jnp.float32)],
    )(pred, rows, ba9, btrue, tgtf)


def kernel(pred, target, kernel, bias):
    sampled_ids, adj_s = _sampled_ids_and_adj()
    tgt = target.reshape(-1).astype(jnp.int32)

    ids = jnp.concatenate([sampled_ids, tgt])
    adj_ext = jnp.concatenate([adj_s, jnp.zeros((BATCH,), jnp.float32)])
    eye = jnp.eye(DIM, 128, dtype=jnp.float32)
    table_lin = _transpad(kernel, eye).reshape(8 * UNITS, DIM)
    rows, ba = _sc_gather(table_lin, bias, ids, ids * 8, adj_ext)

    pred_t1 = jnp.concatenate(
        [pred.T, jnp.ones((1, BATCH), jnp.float32)], axis=0)
    out_t = _projection_t(kernel, bias.reshape(1, UNITS), pred_t1)

    ba9 = ba[:NUM_SAMPLED].reshape(_N_SAMP_BLKS, 1, BATCH)
    btrue = ba[NUM_SAMPLED:].reshape(BATCH, 1)
    tgtf = tgt.astype(jnp.float32).reshape(BATCH, 1)
    loss = _nce_loss(pred, rows, ba9, btrue, tgtf)

    return (out_t.T, loss.reshape(()))


# speedup vs baseline: 1.0093x; 1.0048x over previous
import functools
import math

import jax
import jax.numpy as jnp
from jax import lax
from jax.experimental import pallas as pl
from jax.experimental.pallas import tpu as pltpu
from jax.experimental.pallas import tpu_sc as plsc

UNITS = 100000
NUM_SAMPLED = 8192
BATCH = 1024
DIM = 16
TOTAL_IDS = NUM_SAMPLED + BATCH
NUM_WORKERS = 32
PER_W = TOTAL_IDS // NUM_WORKERS
CHUNKS = 3
CHUNK = PER_W // CHUNKS
LANE = 16
BC = 4096
TBC = 16384
LOG_RANGE = math.log(float(UNITS) + 1.0)


def _sampled_ids_and_adj():
    key = jax.random.key(42)
    u = jax.random.uniform(key, (NUM_SAMPLED,), dtype=jnp.float32)
    s = jnp.exp(u * jnp.log(float(UNITS) + 1.0)) - 1.0
    ids = jnp.clip(s.astype(jnp.int32), 0, UNITS - 1)
    idf = ids.astype(jnp.float32)
    p = (jnp.log(idf + 2.0) - jnp.log(idf + 1.0)) / LOG_RANGE
    adj = jnp.log(-jnp.expm1(float(NUM_SAMPLED) * jnp.log1p(-p)))
    return ids, adj


def _transpad_body(k_ref, eye_ref, out_ref):
    out_ref[...] = lax.dot_general(
        k_ref[...], eye_ref[...], (((0,), (0,)), ((), ())),
        preferred_element_type=jnp.float32)


def _transpad(kern, eye):
    return pl.pallas_call(
        _transpad_body,
        grid=(pl.cdiv(UNITS, TBC),),
        in_specs=[
            pl.BlockSpec((DIM, TBC), lambda j: (0, j)),
            pl.BlockSpec((DIM, 128), lambda j: (0, 0)),
        ],
        out_specs=pl.BlockSpec((TBC, 128), lambda j: (j, 0)),
        out_shape=jax.ShapeDtypeStruct((UNITS, 128), jnp.float32),
    )(kern, eye)


def _sc_gather(table, bias, idx, idx8, adj):
    mesh = plsc.VectorSubcoreMesh(core_axis_name="c", subcore_axis_name="s")

    @functools.partial(
        pl.kernel,
        mesh=mesh,
        out_type=(jax.ShapeDtypeStruct((TOTAL_IDS, DIM), jnp.float32),
                  jax.ShapeDtypeStruct((TOTAL_IDS,), jnp.float32)),
        name="sc_gather",
        scratch_types=[
            pltpu.VMEM((CHUNKS, CHUNK), jnp.int32),
            pltpu.VMEM((CHUNKS, CHUNK), jnp.int32),
            pltpu.VMEM((CHUNKS, CHUNK, DIM), jnp.float32),
            pltpu.VMEM((CHUNKS, CHUNK), jnp.float32),
            pltpu.VMEM((CHUNKS, CHUNK), jnp.float32),
            pltpu.SemaphoreType.DMA,
            pltpu.SemaphoreType.DMA,
        ],
        compiler_params=pltpu.CompilerParams(use_tc_tiling_on_sc=False),
    )
    def gather_kernel(table_hbm, bias_hbm, idx_hbm, idx8_hbm, adj_hbm,
                      out_hbm, bout_hbm,
                      idx_v, idx8_v, rows_v, bias_v, adj_v, isem, sem):
        wid = lax.axis_index("s") * 2 + lax.axis_index("c")
        base = wid * PER_W
        icopies = [pltpu.async_copy(idx_hbm.at[pl.ds(base + j * CHUNK, CHUNK)],
                                    idx_v.at[j], isem) for j in range(CHUNKS)]
        icopies += [pltpu.async_copy(idx8_hbm.at[pl.ds(base + j * CHUNK, CHUNK)],
                                     idx8_v.at[j], isem) for j in range(CHUNKS)]
        icopies += [pltpu.async_copy(adj_hbm.at[pl.ds(base + j * CHUNK, CHUNK)],
                                     adj_v.at[j], isem) for j in range(CHUNKS)]
        for c in icopies:
            c.wait()
        gathers = [pltpu.async_copy(table_hbm.at[idx8_v.at[j]], rows_v.at[j], sem)
                   for j in range(CHUNKS)]
        gathers += [pltpu.async_copy(bias_hbm.at[idx_v.at[j]], bias_v.at[j], sem)
                    for j in range(CHUNKS)]
        for c in gathers:
            c.wait()
        for j in range(CHUNKS):
            for k in range(CHUNK // LANE):
                sl = pl.ds(k * LANE, LANE)
                bias_v[j, sl] = bias_v[j, sl] - adj_v[j, sl]
        wcopies = [pltpu.async_copy(rows_v.at[j],
                                    out_hbm.at[pl.ds(base + j * CHUNK, CHUNK)],
                                    isem) for j in range(CHUNKS)]
        wcopies += [pltpu.async_copy(bias_v.at[j],
                                     bout_hbm.at[pl.ds(base + j * CHUNK, CHUNK)],
                                     isem) for j in range(CHUNKS)]
        for c in wcopies:
            c.wait()

    return gather_kernel(table, bias, idx, idx8, adj)


def _mm_body(k_ref, b_ref, pt_ref, out_ref):
    kb = jnp.concatenate([k_ref[...], b_ref[...]], axis=0)
    out_ref[...] = lax.dot_general(
        kb, pt_ref[...], (((0,), (0,)), ((), ())),
        preferred_element_type=jnp.float32)


def _projection_t(kern, bias_row, pred_t1):
    grid = (pl.cdiv(UNITS, BC),)
    return pl.pallas_call(
        _mm_body,
        grid=grid,
        in_specs=[
            pl.BlockSpec((DIM, BC), lambda j: (0, j)),
            pl.BlockSpec((1, BC), lambda j: (0, j)),
            pl.BlockSpec((DIM + 1, BATCH), lambda j: (0, 0)),
        ],
        out_specs=pl.BlockSpec((BC, BATCH), lambda j: (j, 0)),
        out_shape=jax.ShapeDtypeStruct((UNITS, BATCH), jnp.float32),
    )(kern, bias_row, pred_t1)


_N_SAMP_BLKS = NUM_SAMPLED // BATCH


def _loss_body(pred_ref, rows_ref, ba_ref, bt_ref, tgt_ref, out_ref, acc_ref):
    j = pl.program_id(0)

    @pl.when(j == 0)
    def _init():
        acc_ref[...] = jnp.zeros_like(acc_ref)

    @pl.when(j < _N_SAMP_BLKS)
    def _sampled():
        logits = lax.dot_general(pred_ref[...], rows_ref[...],
                                 (((1,), (1,)), ((), ())),
                                 preferred_element_type=jnp.float32)
        l = logits + ba_ref[0]
        ce = jnp.maximum(l, 0.0) + jnp.log1p(jnp.exp(-jnp.abs(l)))
        ones = jnp.ones((BATCH, 1), jnp.float32)
        acc_ref[...] += lax.dot_general(
            ce, ones, (((1,), (0,)), ((), ())),
            preferred_element_type=jnp.float32)

    @pl.when(j == _N_SAMP_BLKS)
    def _true():
        tl = jnp.sum(pred_ref[...] * rows_ref[...], axis=1,
                     keepdims=True) + bt_ref[...]
        t = tgt_ref[...]
        p = (jnp.log(t + 2.0) - jnp.log(t + 1.0)) / LOG_RANGE
        ec = 1.0 - jnp.exp(float(NUM_SAMPLED) * jnp.log1p(-p))
        l = tl - jnp.log(ec)
        ce1 = jnp.maximum(l, 0.0) - l + jnp.log1p(jnp.exp(-jnp.abs(l)))
        total = acc_ref[...] + ce1
        out_ref[...] = (jnp.sum(total) / float(BATCH)).reshape(1, 1)


def _nce_loss(pred, rows, ba9, btrue, tgtf):
    return pl.pallas_call(
        _loss_body,
        grid=(_N_SAMP_BLKS + 1,),
        in_specs=[
            pl.BlockSpec((BATCH, DIM), lambda j: (0, 0)),
            pl.BlockSpec((BATCH, DIM), lambda j: (j, 0)),
            pl.BlockSpec((1, 1, BATCH),
                         lambda j: (jnp.minimum(j, _N_SAMP_BLKS - 1), 0, 0)),
            pl.BlockSpec((BATCH, 1), lambda j: (0, 0)),
            pl.BlockSpec((BATCH, 1), lambda j: (0, 0)),
        ],
        out_specs=pl.BlockSpec((1, 1), lambda j: (0, 0)),
        out_shape=jax.ShapeDtypeStruct((1, 1), jnp.float32),
        scratch_shapes=[pltpu.VMEM((BATCH, 1), jnp.float32)],
    )(pred, rows, ba9, btrue, tgtf)


def kernel(pred, target, kernel, bias):
    sampled_ids, adj_s = _sampled_ids_and_adj()
    tgt = target.reshape(-1).astype(jnp.int32)

    ids = jnp.concatenate([sampled_ids, tgt])
    adj_ext = jnp.concatenate([adj_s, jnp.zeros((BATCH,), jnp.float32)])
    eye = jnp.eye(DIM, 128, dtype=jnp.float32)
    table_lin = _transpad(kernel, eye).reshape(8 * UNITS, DIM)
    rows, ba = _sc_gather(table_lin, bias, ids, ids * 8, adj_ext)

    pred_t1 = jnp.concatenate(
        [pred.T, jnp.ones((1, BATCH), jnp.float32)], axis=0)
    out_t = _projection_t(kernel, bias.reshape(1, UNITS), pred_t1)

    ba9 = ba[:NUM_SAMPLED].reshape(_N_SAMP_BLKS, 1, BATCH)
    btrue = ba[NUM_SAMPLED:].reshape(BATCH, 1)
    tgtf = tgt.astype(jnp.float32).reshape(BATCH, 1)
    loss = _nce_loss(pred, rows, ba9, btrue, tgtf)

    return (out_t.T, loss.reshape(()))
